# baseline (speedup 1.0000x reference)
import functools

import jax
import jax.numpy as jnp
from jax import lax
from jax.experimental import pallas as pl
from jax.experimental.pallas import tpu as pltpu
from jax.experimental.pallas import tpu_sc as plsc

NUM_CLASSES = 10
N_PROXY = 16
K = 5
TEMP = 0.05
EPSILON = 0.5
SINKHORN_ITERS = 5
FEAT_DIM = 128
BATCH = 16384

ROW_TILE = 2048
NUM_TILES = BATCH // ROW_TILE

NSUB = 16
CHUNK = BATCH // NSUB
NPX = NUM_CLASSES * N_PROXY


def _rowsum(x):
    return jnp.sum(x, axis=1, keepdims=True)


def _row_lse(x):
    m = jnp.max(x, axis=1, keepdims=True)
    return m + jnp.log(_rowsum(jnp.exp(x - m)))


def _sub_sum(x):
    return jnp.sum(x, axis=0, keepdims=True)


def _sub_lse(x):
    m = jnp.max(x, axis=0, keepdims=True)
    return m + jnp.log(_sub_sum(jnp.exp(x - m)))


def _stage_a_body(f_ref, oh_ref, proxy_ref, c_out, l_out, neg_out, pc_out,
                  inv_out, cnt_sc):
    i = pl.program_id(0)
    f = f_ref[...]
    nrm = jnp.sqrt(_rowsum(f * f))
    fn = f / jnp.maximum(nrm, 1e-12)
    g_t = lax.dot_general(proxy_ref[...], fn, (((1,), (1,)), ((), ())),
                          preferred_element_type=jnp.float32)
    oh = oh_ref[...]
    c_t = jnp.zeros((N_PROXY, ROW_TILE), dtype=jnp.float32)
    for c in range(NUM_CLASSES):
        c_t = c_t + oh[c:c + 1, :] * g_t[c * N_PROXY:(c + 1) * N_PROXY, :]
    x = c_t * (1.0 / TEMP)
    l_t = x - _sub_lse(x)
    neg_out[...] = _sub_lse(l_t)

    ones_t = jnp.ones((1, ROW_TILE), dtype=jnp.float32)
    cnt_tile = lax.dot_general(ones_t, oh, (((1,), (1,)), ((), ())),
                               preferred_element_type=jnp.float32)
    prev = jnp.where(i == 0, jnp.zeros((1, NUM_CLASSES), jnp.float32),
                     cnt_sc[:, 0:NUM_CLASSES])
    cnt = prev + cnt_tile
    cnt_sc[:, 0:NUM_CLASSES] = cnt

    @pl.when(i == NUM_TILES - 1)
    def _():
        full = jnp.concatenate(
            [cnt, jnp.zeros((1, N_PROXY - NUM_CLASSES), jnp.float32)], axis=1)
        inv_out[...] = 1.0 / jnp.where(full > 0, full, 1.0)

    eye = (lax.broadcasted_iota(jnp.int32, (N_PROXY, N_PROXY), 0) ==
           lax.broadcasted_iota(jnp.int32, (N_PROXY, N_PROXY), 1)
           ).astype(jnp.float32)
    c_out[...] = lax.dot_general(c_t, eye, (((0,), (0,)), ((), ())),
                                 preferred_element_type=jnp.float32)
    l_out[...] = lax.dot_general(l_t, eye, (((0,), (0,)), ((), ())),
                                 preferred_element_type=jnp.float32)

    @pl.when(i == 0)
    def _():
        P = proxy_ref[...]
        sim = jnp.clip(lax.dot_general(P, P, (((1,), (1,)), ((), ())),
                                       preferred_element_type=jnp.float32)
                       * (1.0 / TEMP), -10.0, 10.0)
        rown = lax.broadcasted_iota(jnp.int32, (NPX, N_PROXY), 0)
        coln = lax.broadcasted_iota(jnp.int32, (NPX, N_PROXY), 1)
        acc = jnp.zeros((NPX, 1), dtype=jnp.float32)
        ldiag = jnp.zeros((NPX, 1), dtype=jnp.float32)
        for c2 in range(NUM_CLASSES):
            xb = sim[:, c2 * N_PROXY:(c2 + 1) * N_PROXY]
            lb = xb - _row_lse(xb)
            acc = acc + _row_lse(lb)
            own = (rown // N_PROXY == c2) & (rown % N_PROXY == coln)
            ldiag = ldiag + _rowsum(jnp.where(own, lb, 0.0))
        pc = jnp.sum(acc - ldiag) * (1.0 / N_PROXY)
        pc_out[...] = jnp.broadcast_to(pc, (1, 1)).astype(jnp.float32)


_stage_a = pl.pallas_call(
    _stage_a_body,
    grid=(NUM_TILES,),
    in_specs=[
        pl.BlockSpec((ROW_TILE, FEAT_DIM), lambda i: (i, 0)),
        pl.BlockSpec((NUM_CLASSES, ROW_TILE), lambda i: (0, i)),
        pl.BlockSpec((NPX, FEAT_DIM), lambda i: (0, 0)),
    ],
    out_specs=[
        pl.BlockSpec((ROW_TILE, N_PROXY), lambda i: (i, 0)),
        pl.BlockSpec((ROW_TILE, N_PROXY), lambda i: (i, 0)),
        pl.BlockSpec((1, ROW_TILE), lambda i: (0, i)),
        pl.BlockSpec((1, 1), lambda i: (0, 0)),
        pl.BlockSpec((1, N_PROXY), lambda i: (0, 0)),
    ],
    out_shape=[
        jax.ShapeDtypeStruct((BATCH, N_PROXY), jnp.float32),
        jax.ShapeDtypeStruct((BATCH, N_PROXY), jnp.float32),
        jax.ShapeDtypeStruct((1, BATCH), jnp.float32),
        jax.ShapeDtypeStruct((1, 1), jnp.float32),
        jax.ShapeDtypeStruct((1, N_PROXY), jnp.float32),
    ],
    scratch_shapes=[pltpu.VMEM((1, N_PROXY), jnp.float32)],
)


def _stage_b_body(c_hbm, l_hbm, t_hbm, neg_hbm, inv_hbm, pc_hbm, out_hbm,
                  cw, lw, qw, tw, uw, negw, invc, pcv, accw, outw,
                  v_ref, vacc_ref, tmp_ref, tmp2_ref, shared, shared2):
    wid = lax.axis_index("s") * 1 + lax.axis_index("c")
    base = wid * CHUNK
    pltpu.sync_copy(c_hbm.at[pl.ds(base * N_PROXY, CHUNK * N_PROXY)], cw)
    pltpu.sync_copy(l_hbm.at[pl.ds(base * N_PROXY, CHUNK * N_PROXY)], lw)
    pltpu.sync_copy(t_hbm.at[pl.ds(base, CHUNK)], tw)
    pltpu.sync_copy(neg_hbm.at[pl.ds(base, CHUNK)], negw)
    pltpu.sync_copy(inv_hbm, invc)
    pltpu.sync_copy(pc_hbm, pcv)

    iota16 = lax.iota(jnp.int32, N_PROXY)
    ones16 = jnp.ones((N_PROXY,), dtype=jnp.float32)
    zeros16 = jnp.zeros((N_PROXY,), dtype=jnp.float32)
    for k in range(NUM_CLASSES):
        v_ref[pl.ds(N_PROXY * k, N_PROXY)] = ones16

    NG = CHUNK // 16

    def _reduce_shared(dst_transform):
        pltpu.sync_copy(vacc_ref, shared.at[wid])
        plsc.subcore_barrier()
        pltpu.sync_copy(shared, tmp_ref)
        plsc.subcore_barrier()
        for k in range(NUM_CLASSES):
            acc = tmp_ref[0, pl.ds(N_PROXY * k, N_PROXY)]
            for j in range(1, NSUB):
                acc = acc + tmp_ref[j, pl.ds(N_PROXY * k, N_PROXY)]
            dst_transform(k, acc)

    def sink_iteration(_it, carry):
        for k in range(NUM_CLASSES):
            vacc_ref[pl.ds(N_PROXY * k, N_PROXY)] = zeros16

        @plsc.parallel_loop(0, NG, unroll=2)
        def sink_body(g):
            tvec = tw[pl.ds(g * 16, 16)]
            uvec = zeros16
            for j in range(16):
                t = tvec[j]
                idx = t * N_PROXY + iota16
                vv = plsc.load_gather(v_ref, [idx])
                cs = cw[pl.ds((g * 16 + j) * N_PROXY, N_PROXY)]
                r = jnp.broadcast_to(jnp.sum(cs * vv), (N_PROXY,))
                u = ones16 / jnp.maximum(r, 1e-10)
                uvec = jnp.where(iota16 == j, u, uvec)
                plsc.addupdate_scatter(vacc_ref, [idx], cs * u)
            uw[pl.ds(g * 16, 16)] = uvec

        def _upd_v(k, acc):
            v_ref[pl.ds(N_PROXY * k, N_PROXY)] = \
                1.0 / jnp.maximum(acc, 1e-10)

        _reduce_shared(_upd_v)
        return carry

    lax.fori_loop(0, SINKHORN_ITERS, sink_iteration, 0)

    for k in range(NUM_CLASSES):
        vacc_ref[pl.ds(N_PROXY * k, N_PROXY)] = zeros16

    @plsc.parallel_loop(0, NG, unroll=2)
    def q_body(g):
        tvec = tw[pl.ds(g * 16, 16)]
        uvec = uw[pl.ds(g * 16, 16)]
        for j in range(16):
            t = tvec[j]
            idx = t * N_PROXY + iota16
            vv = plsc.load_gather(v_ref, [idx])
            cs = cw[pl.ds((g * 16 + j) * N_PROXY, N_PROXY)]
            q = uvec[j] * jnp.exp(cs * (1.0 / EPSILON)) * vv
            qw[pl.ds((g * 16 + j) * N_PROXY, N_PROXY)] = q
            plsc.addupdate_scatter(vacc_ref, [idx], q)

    def _upd_s(k, acc):
        s_val = jnp.sum(acc)
        s_val = jnp.where(s_val > 0, s_val, 1.0)
        v_ref[pl.ds(N_PROXY * k, N_PROXY)] = s_val * ones16

    _reduce_shared(_upd_s)

    @plsc.parallel_loop(0, NG, unroll=2, carry=zeros16)
    def mask_body(g, acc):
        tvec = tw[pl.ds(g * 16, 16)]
        posvec = zeros16
        for j in range(16):
            t = tvec[j]
            idx = t * N_PROXY + iota16
            sv = plsc.load_gather(v_ref, [idx])
            w = qw[pl.ds((g * 16 + j) * N_PROXY, N_PROXY)] / sv
            sk = lax.sort(w)
            thr = jnp.sum(jnp.where(iota16 == N_PROXY - K, sk, 0.0))
            gt = w > thr
            ngt = plsc.all_reduce_population_count(gt)
            eq = w == thr
            eqr = plsc.cumsum(eq.astype(jnp.int32))
            km = gt | (eq & ((eqr + ngt) <= K))
            ls = lw[pl.ds((g * 16 + j) * N_PROXY, N_PROXY)]
            pos = jnp.sum(jnp.where(km, w * ls, 0.0))
            posvec = jnp.where(iota16 == j, pos, posvec)
        negvec = negw[pl.ds(g * 16, 16)]
        wv = plsc.load_gather(invc, [tvec])
        return acc + (posvec - negvec) * wv

    accw[pl.ds(0, N_PROXY)] = mask_body
    pltpu.sync_copy(accw, shared2.at[wid])
    plsc.subcore_barrier()

    @pl.when(wid == 0)
    def _():
        pltpu.sync_copy(shared2, tmp2_ref)
        tot = tmp2_ref[0, pl.ds(0, N_PROXY)]
        for j in range(1, NSUB):
            tot = tot + tmp2_ref[j, pl.ds(0, N_PROXY)]
        mle = jnp.broadcast_to(-jnp.sum(tot), (N_PROXY,))
        pcvec = pcv[pl.ds(0, N_PROXY)]
        total = (mle + pcvec) / jnp.full((N_PROXY,), float(NUM_CLASSES),
                                         jnp.float32)
        outw[pl.ds(0, N_PROXY)] = total
        pltpu.sync_copy(outw, out_hbm)


_stage_b = functools.partial(
    pl.kernel,
    out_type=jax.ShapeDtypeStruct((N_PROXY,), jnp.float32),
    mesh=plsc.VectorSubcoreMesh(core_axis_name="c", subcore_axis_name="s",
                                num_cores=1),
    compiler_params=pltpu.CompilerParams(needs_layout_passes=False),
    scratch_types=[
        pltpu.VMEM((CHUNK * N_PROXY,), jnp.float32),
        pltpu.VMEM((CHUNK * N_PROXY,), jnp.float32),
        pltpu.VMEM((CHUNK * N_PROXY,), jnp.float32),
        pltpu.VMEM((CHUNK,), jnp.int32),
        pltpu.VMEM((CHUNK,), jnp.float32),
        pltpu.VMEM((CHUNK,), jnp.float32),
        pltpu.VMEM((N_PROXY,), jnp.float32),
        pltpu.VMEM((N_PROXY,), jnp.float32),
        pltpu.VMEM((N_PROXY,), jnp.float32),
        pltpu.VMEM((N_PROXY,), jnp.float32),
        pltpu.VMEM((NPX,), jnp.float32),
        pltpu.VMEM((NPX,), jnp.float32),
        pltpu.VMEM((NSUB, NPX), jnp.float32),
        pltpu.VMEM((NSUB, N_PROXY), jnp.float32),
        pltpu.VMEM_SHARED((NSUB, NPX), jnp.float32),
        pltpu.VMEM_SHARED((NSUB, N_PROXY), jnp.float32),
    ],
)(_stage_b_body)


@jax.jit
def kernel(features, targets, proxy):
    oh = (targets[None, :] ==
          jnp.arange(NUM_CLASSES, dtype=targets.dtype)[:, None]
          ).astype(jnp.float32)
    c_rows, l_rows, neg, pc, inv = _stage_a(features, oh, proxy)
    out = _stage_b(c_rows.reshape(-1), l_rows.reshape(-1), targets,
                   neg.reshape(-1), inv.reshape(-1),
                   jnp.broadcast_to(pc.reshape(1), (N_PROXY,)))
    return out[0]


# speedup vs baseline: 1.0499x; 1.0499x over previous
import functools

import jax
import jax.numpy as jnp
from jax import lax
from jax.experimental import pallas as pl
from jax.experimental.pallas import tpu as pltpu
from jax.experimental.pallas import tpu_sc as plsc

NUM_CLASSES = 10
N_PROXY = 16
K = 5
TEMP = 0.05
EPSILON = 0.5
SINKHORN_ITERS = 5
FEAT_DIM = 128
BATCH = 16384

ROW_TILE = 2048
NUM_TILES = BATCH // ROW_TILE

NSUB = 16
CHUNK = BATCH // NSUB
NPX = NUM_CLASSES * N_PROXY


def _rowsum(x):
    return jnp.sum(x, axis=1, keepdims=True)


def _row_lse(x):
    m = jnp.max(x, axis=1, keepdims=True)
    return m + jnp.log(_rowsum(jnp.exp(x - m)))


def _sub_sum(x):
    return jnp.sum(x, axis=0, keepdims=True)


def _sub_lse(x):
    m = jnp.max(x, axis=0, keepdims=True)
    return m + jnp.log(_sub_sum(jnp.exp(x - m)))


def _stage_a_body(f_ref, oh_ref, proxy_ref, c_out, l_out, neg_out, pc_out,
                  inv_out, cnt_sc):
    i = pl.program_id(0)
    f = f_ref[...]
    nrm = jnp.sqrt(_rowsum(f * f))
    fn = f / jnp.maximum(nrm, 1e-12)
    g_t = lax.dot_general(proxy_ref[...], fn, (((1,), (1,)), ((), ())),
                          preferred_element_type=jnp.float32)
    oh = oh_ref[...]
    c_t = jnp.zeros((N_PROXY, ROW_TILE), dtype=jnp.float32)
    for c in range(NUM_CLASSES):
        c_t = c_t + oh[c:c + 1, :] * g_t[c * N_PROXY:(c + 1) * N_PROXY, :]
    x = c_t * (1.0 / TEMP)
    l_t = x - _sub_lse(x)
    neg_out[...] = _sub_lse(l_t)

    ones_t = jnp.ones((1, ROW_TILE), dtype=jnp.float32)
    cnt_tile = lax.dot_general(ones_t, oh, (((1,), (1,)), ((), ())),
                               preferred_element_type=jnp.float32)
    prev = jnp.where(i == 0, jnp.zeros((1, NUM_CLASSES), jnp.float32),
                     cnt_sc[:, 0:NUM_CLASSES])
    cnt = prev + cnt_tile
    cnt_sc[:, 0:NUM_CLASSES] = cnt

    @pl.when(i == NUM_TILES - 1)
    def _():
        full = jnp.concatenate(
            [cnt, jnp.zeros((1, N_PROXY - NUM_CLASSES), jnp.float32)], axis=1)
        inv_out[...] = 1.0 / jnp.where(full > 0, full, 1.0)

    eye = (lax.broadcasted_iota(jnp.int32, (N_PROXY, N_PROXY), 0) ==
           lax.broadcasted_iota(jnp.int32, (N_PROXY, N_PROXY), 1)
           ).astype(jnp.float32)
    c_out[...] = lax.dot_general(c_t, eye, (((0,), (0,)), ((), ())),
                                 preferred_element_type=jnp.float32)
    l_out[...] = lax.dot_general(l_t, eye, (((0,), (0,)), ((), ())),
                                 preferred_element_type=jnp.float32)

    @pl.when(i == 0)
    def _():
        P = proxy_ref[...]
        sim = jnp.clip(lax.dot_general(P, P, (((1,), (1,)), ((), ())),
                                       preferred_element_type=jnp.float32)
                       * (1.0 / TEMP), -10.0, 10.0)
        rown = lax.broadcasted_iota(jnp.int32, (NPX, N_PROXY), 0)
        coln = lax.broadcasted_iota(jnp.int32, (NPX, N_PROXY), 1)
        acc = jnp.zeros((NPX, 1), dtype=jnp.float32)
        ldiag = jnp.zeros((NPX, 1), dtype=jnp.float32)
        for c2 in range(NUM_CLASSES):
            xb = sim[:, c2 * N_PROXY:(c2 + 1) * N_PROXY]
            lb = xb - _row_lse(xb)
            acc = acc + _row_lse(lb)
            own = (rown // N_PROXY == c2) & (rown % N_PROXY == coln)
            ldiag = ldiag + _rowsum(jnp.where(own, lb, 0.0))
        pc = jnp.sum(acc - ldiag) * (1.0 / N_PROXY)
        pc_out[...] = jnp.broadcast_to(pc, (1, 1)).astype(jnp.float32)


_stage_a = pl.pallas_call(
    _stage_a_body,
    grid=(NUM_TILES,),
    in_specs=[
        pl.BlockSpec((ROW_TILE, FEAT_DIM), lambda i: (i, 0)),
        pl.BlockSpec((NUM_CLASSES, ROW_TILE), lambda i: (0, i)),
        pl.BlockSpec((NPX, FEAT_DIM), lambda i: (0, 0)),
    ],
    out_specs=[
        pl.BlockSpec((ROW_TILE, N_PROXY), lambda i: (i, 0)),
        pl.BlockSpec((ROW_TILE, N_PROXY), lambda i: (i, 0)),
        pl.BlockSpec((1, ROW_TILE), lambda i: (0, i)),
        pl.BlockSpec((1, 1), lambda i: (0, 0)),
        pl.BlockSpec((1, N_PROXY), lambda i: (0, 0)),
    ],
    out_shape=[
        jax.ShapeDtypeStruct((BATCH, N_PROXY), jnp.float32),
        jax.ShapeDtypeStruct((BATCH, N_PROXY), jnp.float32),
        jax.ShapeDtypeStruct((1, BATCH), jnp.float32),
        jax.ShapeDtypeStruct((1, 1), jnp.float32),
        jax.ShapeDtypeStruct((1, N_PROXY), jnp.float32),
    ],
    scratch_shapes=[pltpu.VMEM((1, N_PROXY), jnp.float32)],
)


def _stage_b_body(c_hbm, l_hbm, t_hbm, neg_hbm, inv_hbm, pc_hbm, out_hbm,
                  cw, lw, qw, tw, uw, negw, invc, pcv, accw, outw,
                  v_ref, vacc_ref, tmp_ref, tmp2_ref, shared, shared2):
    wid = lax.axis_index("s") * 1 + lax.axis_index("c")
    base = wid * CHUNK
    pltpu.sync_copy(c_hbm.at[pl.ds(base * N_PROXY, CHUNK * N_PROXY)], cw)
    pltpu.sync_copy(l_hbm.at[pl.ds(base * N_PROXY, CHUNK * N_PROXY)], lw)
    pltpu.sync_copy(t_hbm.at[pl.ds(base, CHUNK)], tw)
    pltpu.sync_copy(neg_hbm.at[pl.ds(base, CHUNK)], negw)
    pltpu.sync_copy(inv_hbm, invc)
    pltpu.sync_copy(pc_hbm, pcv)

    iota16 = lax.iota(jnp.int32, N_PROXY)
    ones16 = jnp.ones((N_PROXY,), dtype=jnp.float32)
    zeros16 = jnp.zeros((N_PROXY,), dtype=jnp.float32)
    for k in range(NUM_CLASSES):
        v_ref[pl.ds(N_PROXY * k, N_PROXY)] = ones16

    NG = CHUNK // 16

    def _reduce_shared(dst_transform):
        pltpu.sync_copy(vacc_ref, shared.at[wid])
        plsc.subcore_barrier()
        pltpu.sync_copy(shared, tmp_ref)
        plsc.subcore_barrier()
        for k in range(NUM_CLASSES):
            acc = tmp_ref[0, pl.ds(N_PROXY * k, N_PROXY)]
            for j in range(1, NSUB):
                acc = acc + tmp_ref[j, pl.ds(N_PROXY * k, N_PROXY)]
            dst_transform(k, acc)

    def sink_iteration(_it, carry):
        for k in range(NUM_CLASSES):
            vacc_ref[pl.ds(N_PROXY * k, N_PROXY)] = zeros16

        @plsc.parallel_loop(0, NG, unroll=4)
        def sink_body(g):
            tvec = tw[pl.ds(g * 16, 16)]
            uvec = zeros16
            for j in range(16):
                t = tvec[j]
                idx = t * N_PROXY + iota16
                vv = plsc.load_gather(v_ref, [idx])
                cs = cw[pl.ds((g * 16 + j) * N_PROXY, N_PROXY)]
                r = jnp.broadcast_to(jnp.sum(cs * vv), (N_PROXY,))
                u = ones16 / jnp.maximum(r, 1e-10)
                uvec = jnp.where(iota16 == j, u, uvec)
                plsc.addupdate_scatter(vacc_ref, [idx], cs * u)
            uw[pl.ds(g * 16, 16)] = uvec

        def _upd_v(k, acc):
            v_ref[pl.ds(N_PROXY * k, N_PROXY)] = \
                1.0 / jnp.maximum(acc, 1e-10)

        _reduce_shared(_upd_v)
        return carry

    lax.fori_loop(0, SINKHORN_ITERS, sink_iteration, 0)

    for k in range(NUM_CLASSES):
        vacc_ref[pl.ds(N_PROXY * k, N_PROXY)] = zeros16

    @plsc.parallel_loop(0, NG, unroll=4)
    def q_body(g):
        tvec = tw[pl.ds(g * 16, 16)]
        uvec = uw[pl.ds(g * 16, 16)]
        for j in range(16):
            t = tvec[j]
            idx = t * N_PROXY + iota16
            vv = plsc.load_gather(v_ref, [idx])
            cs = cw[pl.ds((g * 16 + j) * N_PROXY, N_PROXY)]
            q = uvec[j] * jnp.exp(cs * (1.0 / EPSILON)) * vv
            qw[pl.ds((g * 16 + j) * N_PROXY, N_PROXY)] = q
            plsc.addupdate_scatter(vacc_ref, [idx], q)

    def _upd_s(k, acc):
        s_val = jnp.sum(acc)
        s_val = jnp.where(s_val > 0, s_val, 1.0)
        v_ref[pl.ds(N_PROXY * k, N_PROXY)] = s_val * ones16

    _reduce_shared(_upd_s)

    @plsc.parallel_loop(0, NG, unroll=2, carry=zeros16)
    def mask_body(g, acc):
        tvec = tw[pl.ds(g * 16, 16)]
        posvec = zeros16
        for j in range(16):
            t = tvec[j]
            idx = t * N_PROXY + iota16
            sv = plsc.load_gather(v_ref, [idx])
            w = qw[pl.ds((g * 16 + j) * N_PROXY, N_PROXY)] / sv
            sk = lax.sort(w)
            thr = jnp.sum(jnp.where(iota16 == N_PROXY - K, sk, 0.0))
            gt = w > thr
            ngt = plsc.all_reduce_population_count(gt)
            eq = w == thr
            eqr = plsc.cumsum(eq.astype(jnp.int32))
            km = gt | (eq & ((eqr + ngt) <= K))
            ls = lw[pl.ds((g * 16 + j) * N_PROXY, N_PROXY)]
            pos = jnp.sum(jnp.where(km, w * ls, 0.0))
            posvec = jnp.where(iota16 == j, pos, posvec)
        negvec = negw[pl.ds(g * 16, 16)]
        wv = plsc.load_gather(invc, [tvec])
        return acc + (posvec - negvec) * wv

    accw[pl.ds(0, N_PROXY)] = mask_body
    pltpu.sync_copy(accw, shared2.at[wid])
    plsc.subcore_barrier()

    @pl.when(wid == 0)
    def _():
        pltpu.sync_copy(shared2, tmp2_ref)
        tot = tmp2_ref[0, pl.ds(0, N_PROXY)]
        for j in range(1, NSUB):
            tot = tot + tmp2_ref[j, pl.ds(0, N_PROXY)]
        mle = jnp.broadcast_to(-jnp.sum(tot), (N_PROXY,))
        pcvec = pcv[pl.ds(0, N_PROXY)]
        total = (mle + pcvec) / jnp.full((N_PROXY,), float(NUM_CLASSES),
                                         jnp.float32)
        outw[pl.ds(0, N_PROXY)] = total
        pltpu.sync_copy(outw, out_hbm)


_stage_b = functools.partial(
    pl.kernel,
    out_type=jax.ShapeDtypeStruct((N_PROXY,), jnp.float32),
    mesh=plsc.VectorSubcoreMesh(core_axis_name="c", subcore_axis_name="s",
                                num_cores=1),
    compiler_params=pltpu.CompilerParams(needs_layout_passes=False),
    scratch_types=[
        pltpu.VMEM((CHUNK * N_PROXY,), jnp.float32),
        pltpu.VMEM((CHUNK * N_PROXY,), jnp.float32),
        pltpu.VMEM((CHUNK * N_PROXY,), jnp.float32),
        pltpu.VMEM((CHUNK,), jnp.int32),
        pltpu.VMEM((CHUNK,), jnp.float32),
        pltpu.VMEM((CHUNK,), jnp.float32),
        pltpu.VMEM((N_PROXY,), jnp.float32),
        pltpu.VMEM((N_PROXY,), jnp.float32),
        pltpu.VMEM((N_PROXY,), jnp.float32),
        pltpu.VMEM((N_PROXY,), jnp.float32),
        pltpu.VMEM((NPX,), jnp.float32),
        pltpu.VMEM((NPX,), jnp.float32),
        pltpu.VMEM((NSUB, NPX), jnp.float32),
        pltpu.VMEM((NSUB, N_PROXY), jnp.float32),
        pltpu.VMEM_SHARED((NSUB, NPX), jnp.float32),
        pltpu.VMEM_SHARED((NSUB, N_PROXY), jnp.float32),
    ],
)(_stage_b_body)


@jax.jit
def kernel(features, targets, proxy):
    oh = (targets[None, :] ==
          jnp.arange(NUM_CLASSES, dtype=targets.dtype)[:, None]
          ).astype(jnp.float32)
    c_rows, l_rows, neg, pc, inv = _stage_a(features, oh, proxy)
    out = _stage_b(c_rows.reshape(-1), l_rows.reshape(-1), targets,
                   neg.reshape(-1), inv.reshape(-1),
                   jnp.broadcast_to(pc.reshape(1), (N_PROXY,)))
    return out[0]
